# baseline (speedup 1.0000x reference)
import functools

import jax
import jax.numpy as jnp
from jax import lax
from jax.experimental import pallas as pl
from jax.experimental.pallas import tpu as pltpu
from jax.experimental.pallas import tpu_sc as plsc

BLK = 2048
LANE = 16
NC = 2
NS = 16
NW = NC * NS


def _make_stage1(B, H, A):
    nblk = pl.cdiv(A, BLK)

    def body(x_ref, w_ref, b_ref, am_ref, iam_ref, om_ref, wcg_ref, bcg_ref,
             e_ref, s_ref, cc_ref, pg_ref, gcol1):
        i = pl.program_id(0)
        xb = x_ref[...]
        gen = jnp.dot(xb, w_ref[...], preferred_element_type=jnp.float32)
        gen = gen + b_ref[...]

        @pl.when(i == 0)
        def _():
            pg = jnp.dot(xb, wcg_ref[...], preferred_element_type=jnp.float32)
            pg_ref[...] = pg + bcg_ref[...]
            gcol1[...] = jnp.broadcast_to(gen[:, 1:2], gcol1.shape)
            s_ref[...] = jnp.zeros_like(s_ref)
            cc_ref[...] = jnp.zeros_like(cc_ref)

        colid = i * BLK + lax.broadcasted_iota(jnp.int32, (1, BLK), 1)
        sel = jnp.where(om_ref[...] == colid, gen, gcol1[:, :1])
        valid = colid < A
        am = am_ref[...]
        e = jnp.where(valid & (am > 0), jnp.exp(sel), 0.0)
        e_ref[...] = e
        spart = jnp.sum(e, axis=1, keepdims=True)
        ccpart = jnp.sum(jnp.where(valid, am * iam_ref[...], 0), axis=1,
                         keepdims=True)
        s_ref[...] += jnp.broadcast_to(spart, s_ref.shape)
        cc_ref[...] += jnp.broadcast_to(ccpart, cc_ref.shape)

    return pl.pallas_call(
        body,
        grid=(nblk,),
        in_specs=[
            pl.BlockSpec((B, H), lambda i: (0, 0)),
            pl.BlockSpec((H, BLK), lambda i: (0, i)),
            pl.BlockSpec((1, BLK), lambda i: (0, i)),
            pl.BlockSpec((B, BLK), lambda i: (0, i)),
            pl.BlockSpec((1, BLK), lambda i: (0, i)),
            pl.BlockSpec((1, BLK), lambda i: (0, i)),
            pl.BlockSpec((H, 128), lambda i: (0, 0)),
            pl.BlockSpec((1, 128), lambda i: (0, 0)),
        ],
        out_specs=[
            pl.BlockSpec((B, BLK), lambda i: (0, i)),
            pl.BlockSpec((B, 128), lambda i: (0, 0)),
            pl.BlockSpec((B, 128), lambda i: (0, 0)),
            pl.BlockSpec((B, 128), lambda i: (0, 0)),
        ],
        out_shape=[
            jax.ShapeDtypeStruct((B, A), jnp.float32),
            jax.ShapeDtypeStruct((B, 128), jnp.float32),
            jax.ShapeDtypeStruct((B, 128), jnp.int32),
            jax.ShapeDtypeStruct((B, 128), jnp.float32),
        ],
        scratch_shapes=[pltpu.VMEM((B, 128), jnp.float32)],
    )


def _make_stage2(B, A, L):
    nblk = pl.cdiv(A, BLK)

    def body(e_ref, s_ref, cc_ref, pg_ref, attn_ref,
             gp_ref, od_ref, pog_ref, wat_ref, p0s, invs):
        i = pl.program_id(0)

        @pl.when(i == 0)
        def _():
            s = s_ref[:, :1]
            c = cc_ref[:, :1]
            pg0 = pg_ref[:, :1]
            pg1 = pg_ref[:, 1:2]
            pg1 = jnp.where(c > 0, pg1, -jnp.inf)
            m = jnp.maximum(pg0, pg1)
            e0 = jnp.exp(pg0 - m)
            e1 = jnp.exp(pg1 - m)
            den = e0 + e1
            p0 = e0 / den
            p1 = e1 / den
            lanei = lax.broadcasted_iota(jnp.int32, (1, 128), 1)
            pog_ref[...] = jnp.where(lanei == 0, p0,
                                     jnp.where(lanei == 1, p1, 0.0))
            wat_ref[...] = p1 * attn_ref[...]
            p0s[...] = jnp.broadcast_to(p0, p0s.shape)
            invs[...] = jnp.broadcast_to(1.0 / s, invs.shape)

        gp = e_ref[...] * invs[:, :1]
        gp_ref[...] = gp
        od_ref[...] = p0s[:, :1] * gp

    return pl.pallas_call(
        body,
        grid=(nblk,),
        in_specs=[
            pl.BlockSpec((B, BLK), lambda i: (0, i)),
            pl.BlockSpec((B, 128), lambda i: (0, 0)),
            pl.BlockSpec((B, 128), lambda i: (0, 0)),
            pl.BlockSpec((B, 128), lambda i: (0, 0)),
            pl.BlockSpec((B, L), lambda i: (0, 0)),
        ],
        out_specs=[
            pl.BlockSpec((B, BLK), lambda i: (0, i)),
            pl.BlockSpec((B, BLK), lambda i: (0, i)),
            pl.BlockSpec((B, 128), lambda i: (0, 0)),
            pl.BlockSpec((B, L), lambda i: (0, 0)),
        ],
        out_shape=[
            jax.ShapeDtypeStruct((B, A), jnp.float32),
            jax.ShapeDtypeStruct((B, A), jnp.float32),
            jax.ShapeDtypeStruct((B, 128), jnp.float32),
            jax.ShapeDtypeStruct((B, L), jnp.float32),
        ],
        scratch_shapes=[pltpu.VMEM((B, 128), jnp.float32),
                        pltpu.VMEM((B, 128), jnp.float32)],
    )


def _make_sc(B, A, V, L):
    rows_per_w = B // NW
    nch = L // LANE
    accn = ((V + 127) // 128) * 128
    mesh = plsc.VectorSubcoreMesh(core_axis_name="c", subcore_axis_name="s",
                                  num_cores=NC, num_subcores=NS)

    @functools.partial(
        pl.kernel, mesh=mesh, out_type=(),
        compiler_params=pltpu.CompilerParams(needs_layout_passes=False),
        scratch_types=[
            pltpu.VMEM((L,), jnp.int32),
            pltpu.VMEM((L,), jnp.float32),
            pltpu.VMEM((L,), jnp.int32),
            pltpu.VMEM((L,), jnp.float32),
            pltpu.VMEM((L,), jnp.int32),
            pltpu.VMEM((L,), jnp.float32),
            pltpu.VMEM((L,), jnp.float32),
            pltpu.VMEM((accn,), jnp.float32),
            pltpu.SemaphoreType.DMA,
        ],
    )
    def sc_kernel(ctx_hbm, wat_hbm, i2a_hbm, win_hbm, zeros_hbm, od_hbm,
                  ctx_v, wat_v, a_v, win_v, flat_v, val_v, oval_v, acc_v,
                  sem):
        cid = lax.axis_index("c")
        sid = lax.axis_index("s")
        wid = sid * NC + cid
        zeros16 = jnp.zeros((LANE,), jnp.float32)
        pltpu.sync_copy(zeros_hbm, acc_v)
        for r in range(rows_per_w):
            b = wid * rows_per_w + r
            pltpu.sync_copy(ctx_hbm.at[b], ctx_v)
            pltpu.sync_copy(wat_hbm.at[b], wat_v)
            pltpu.async_copy(i2a_hbm.at[ctx_v], a_v, sem).wait()
            for c in range(nch):
                s = pl.ds(LANE * c, LANE)
                flat_v[s] = b * A + a_v[s]
            pltpu.async_copy(win_hbm.at[flat_v], win_v, sem).wait()
            for c in range(nch):
                s = pl.ds(LANE * c, LANE)
                plsc.addupdate_scatter(acc_v, [ctx_v[s]], wat_v[s])
            pltpu.async_copy(od_hbm.at[flat_v], oval_v, sem).wait()
            for c in range(nch):
                s = pl.ds(LANE * c, LANE)
                wv16 = win_v[s].astype(jnp.int32)
                val_v[s] = plsc.load_gather(acc_v, [wv16]) + oval_v[s]
            pltpu.sync_copy(val_v, od_hbm.at[flat_v])
            for c in range(nch):
                s = pl.ds(LANE * c, LANE)
                plsc.store_scatter(acc_v, [ctx_v[s]], zeros16)

    return sc_kernel


def kernel(x, attn_probs, ctx_ids, actionmask, W_gen, b_gen, W_cg, b_cg,
           inp_to_act, inp_actmask, out_map):
    B, H = x.shape
    A = W_gen.shape[1]
    V = inp_to_act.shape[0]
    L = attn_probs.shape[1]

    b_gen2 = b_gen.reshape(1, A)
    iam2 = inp_actmask.reshape(1, A)
    om2 = out_map.reshape(1, A).astype(jnp.int32)
    wcg_pad = jnp.pad(W_cg, ((0, 0), (0, 128 - W_cg.shape[1])))
    bcg_pad = jnp.pad(b_cg, (0, 128 - b_cg.shape[0])).reshape(1, 128)

    e, s_acc, cc_acc, pg_raw = _make_stage1(B, H, A)(
        x, W_gen, b_gen2, actionmask, iam2, om2, wcg_pad, bcg_pad)
    gen_probs, out_dense, pog_full, wattn = _make_stage2(B, A, L)(
        e, s_acc, cc_acc, pg_raw, attn_probs)

    rows = jnp.arange(B)[:, None]
    probe_vals = jnp.broadcast_to(
        jnp.arange(V, dtype=jnp.float32)[None, :], (B, V))
    probe_idx = jnp.broadcast_to(inp_to_act[None, :], (B, V))
    winner = jnp.full((B, A), -1.0, jnp.float32).at[rows, probe_idx].set(
        probe_vals).reshape(B * A)

    od_ref = jax.new_ref(out_dense.reshape(B * A))
    accn = ((V + 127) // 128) * 128
    zeros_init = jnp.zeros((accn,), jnp.float32)
    _make_sc(B, A, V, L)(ctx_ids, wattn, inp_to_act, winner, zeros_init,
                         od_ref)
    out_probs = od_ref[...].reshape(B, A)

    return (out_probs, pog_full[:, :2], gen_probs, attn_probs)


# speedup vs baseline: 1.0055x; 1.0055x over previous
import functools

import jax
import jax.numpy as jnp
from jax import lax
from jax.experimental import pallas as pl
from jax.experimental.pallas import tpu as pltpu
from jax.experimental.pallas import tpu_sc as plsc

BLK = 2048
LANE = 16
NC = 2
NS = 16
NW = NC * NS


def _make_stage1(B, H, A):
    nblk = pl.cdiv(A, BLK)

    def body(x_ref, w_ref, b_ref, am_ref, iam_ref, om_ref, wcg_ref, bcg_ref,
             e_ref, s_ref, cc_ref, pg_ref, gcol1):
        i = pl.program_id(0)
        xb = x_ref[...]
        gen = jnp.dot(xb, w_ref[...], preferred_element_type=jnp.float32)
        gen = gen + b_ref[...]

        @pl.when(i == 0)
        def _():
            pg = jnp.dot(xb, wcg_ref[...], preferred_element_type=jnp.float32)
            pg_ref[...] = pg + bcg_ref[...]
            gcol1[...] = jnp.broadcast_to(gen[:, 1:2], gcol1.shape)
            s_ref[...] = jnp.zeros_like(s_ref)
            cc_ref[...] = jnp.zeros_like(cc_ref)

        colid = i * BLK + lax.broadcasted_iota(jnp.int32, (1, BLK), 1)
        sel = jnp.where(om_ref[...] == colid, gen, gcol1[:, :1])
        valid = colid < A
        am = jnp.swapaxes(am_ref[...], 0, 1)
        e = jnp.where(valid & (am > 0), jnp.exp(sel), 0.0)
        e_ref[...] = e
        spart = jnp.sum(e, axis=1, keepdims=True)
        ccpart = jnp.sum(jnp.where(valid, am * iam_ref[...], 0), axis=1,
                         keepdims=True)
        s_ref[...] += jnp.broadcast_to(spart, s_ref.shape)
        cc_ref[...] += jnp.broadcast_to(ccpart, cc_ref.shape)

    return pl.pallas_call(
        body,
        grid=(nblk,),
        in_specs=[
            pl.BlockSpec((B, H), lambda i: (0, 0)),
            pl.BlockSpec((H, BLK), lambda i: (0, i)),
            pl.BlockSpec((1, BLK), lambda i: (0, i)),
            pl.BlockSpec((BLK, B), lambda i: (i, 0)),
            pl.BlockSpec((1, BLK), lambda i: (0, i)),
            pl.BlockSpec((1, BLK), lambda i: (0, i)),
            pl.BlockSpec((H, 128), lambda i: (0, 0)),
            pl.BlockSpec((1, 128), lambda i: (0, 0)),
        ],
        out_specs=[
            pl.BlockSpec((B, BLK), lambda i: (0, i)),
            pl.BlockSpec((B, 128), lambda i: (0, 0)),
            pl.BlockSpec((B, 128), lambda i: (0, 0)),
            pl.BlockSpec((B, 128), lambda i: (0, 0)),
        ],
        out_shape=[
            jax.ShapeDtypeStruct((B, A), jnp.float32),
            jax.ShapeDtypeStruct((B, 128), jnp.float32),
            jax.ShapeDtypeStruct((B, 128), jnp.int32),
            jax.ShapeDtypeStruct((B, 128), jnp.float32),
        ],
        scratch_shapes=[pltpu.VMEM((B, 128), jnp.float32)],
    )


def _make_stage2(B, A, L):
    nblk = pl.cdiv(A, BLK)

    def body(e_ref, s_ref, cc_ref, pg_ref, attn_ref,
             gp_ref, od_ref, pog_ref, wat_ref, p0s, invs):
        i = pl.program_id(0)

        @pl.when(i == 0)
        def _():
            s = s_ref[:, :1]
            c = cc_ref[:, :1]
            pg0 = pg_ref[:, :1]
            pg1 = pg_ref[:, 1:2]
            pg1 = jnp.where(c > 0, pg1, -jnp.inf)
            m = jnp.maximum(pg0, pg1)
            e0 = jnp.exp(pg0 - m)
            e1 = jnp.exp(pg1 - m)
            den = e0 + e1
            p0 = e0 / den
            p1 = e1 / den
            lanei = lax.broadcasted_iota(jnp.int32, (1, 128), 1)
            pog_ref[...] = jnp.where(lanei == 0, p0,
                                     jnp.where(lanei == 1, p1, 0.0))
            wat_ref[...] = p1 * attn_ref[...]
            p0s[...] = jnp.broadcast_to(p0, p0s.shape)
            invs[...] = jnp.broadcast_to(1.0 / s, invs.shape)

        gp = e_ref[...] * invs[:, :1]
        gp_ref[...] = jnp.swapaxes(gp, 0, 1)
        od_ref[...] = jnp.swapaxes(p0s[:, :1] * gp, 0, 1)

    return pl.pallas_call(
        body,
        grid=(nblk,),
        in_specs=[
            pl.BlockSpec((B, BLK), lambda i: (0, i)),
            pl.BlockSpec((B, 128), lambda i: (0, 0)),
            pl.BlockSpec((B, 128), lambda i: (0, 0)),
            pl.BlockSpec((B, 128), lambda i: (0, 0)),
            pl.BlockSpec((B, L), lambda i: (0, 0)),
        ],
        out_specs=[
            pl.BlockSpec((BLK, B), lambda i: (i, 0)),
            pl.BlockSpec((BLK, B), lambda i: (i, 0)),
            pl.BlockSpec((B, 128), lambda i: (0, 0)),
            pl.BlockSpec((B, L), lambda i: (0, 0)),
        ],
        out_shape=[
            jax.ShapeDtypeStruct((A, B), jnp.float32),
            jax.ShapeDtypeStruct((A, B), jnp.float32),
            jax.ShapeDtypeStruct((B, 128), jnp.float32),
            jax.ShapeDtypeStruct((B, L), jnp.float32),
        ],
        scratch_shapes=[pltpu.VMEM((B, 128), jnp.float32),
                        pltpu.VMEM((B, 128), jnp.float32)],
    )


def _make_sc(B, A, V, L):
    rows_per_w = B // NW
    nch = L // LANE
    accn = ((V + 127) // 128) * 128
    mesh = plsc.VectorSubcoreMesh(core_axis_name="c", subcore_axis_name="s",
                                  num_cores=NC, num_subcores=NS)

    @functools.partial(
        pl.kernel, mesh=mesh, out_type=(),
        compiler_params=pltpu.CompilerParams(needs_layout_passes=False),
        scratch_types=[
            pltpu.VMEM((L,), jnp.int32),
            pltpu.VMEM((L,), jnp.float32),
            pltpu.VMEM((L,), jnp.int32),
            pltpu.VMEM((L,), jnp.float32),
            pltpu.VMEM((L,), jnp.int32),
            pltpu.VMEM((L,), jnp.int32),
            pltpu.VMEM((L,), jnp.float32),
            pltpu.VMEM((L,), jnp.float32),
            pltpu.VMEM((accn,), jnp.float32),
            pltpu.SemaphoreType.DMA,
        ],
    )
    def sc_kernel(ctx_hbm, wat_hbm, i2a_hbm, win_hbm, zeros_hbm, od_hbm,
                  ctx_v, wat_v, a_v, win_v, flatw_v, flato_v, val_v, oval_v,
                  acc_v, sem):
        cid = lax.axis_index("c")
        sid = lax.axis_index("s")
        wid = sid * NC + cid
        zeros16 = jnp.zeros((LANE,), jnp.float32)
        pltpu.sync_copy(zeros_hbm, acc_v)
        for r in range(rows_per_w):
            b = wid * rows_per_w + r
            pltpu.sync_copy(ctx_hbm.at[b], ctx_v)
            pltpu.sync_copy(wat_hbm.at[b], wat_v)
            pltpu.async_copy(i2a_hbm.at[ctx_v], a_v, sem).wait()
            for c in range(nch):
                s = pl.ds(LANE * c, LANE)
                a16 = a_v[s]
                flatw_v[s] = b * A + a16
                flato_v[s] = a16 * B + b
            pltpu.async_copy(win_hbm.at[flatw_v], win_v, sem).wait()
            for c in range(nch):
                s = pl.ds(LANE * c, LANE)
                plsc.addupdate_scatter(acc_v, [ctx_v[s]], wat_v[s])
            pltpu.async_copy(od_hbm.at[flato_v], oval_v, sem).wait()
            for c in range(nch):
                s = pl.ds(LANE * c, LANE)
                wv16 = win_v[s].astype(jnp.int32)
                val_v[s] = plsc.load_gather(acc_v, [wv16]) + oval_v[s]
            pltpu.sync_copy(val_v, od_hbm.at[flato_v])
            for c in range(nch):
                s = pl.ds(LANE * c, LANE)
                plsc.store_scatter(acc_v, [ctx_v[s]], zeros16)

    return sc_kernel


def kernel(x, attn_probs, ctx_ids, actionmask, W_gen, b_gen, W_cg, b_cg,
           inp_to_act, inp_actmask, out_map):
    B, H = x.shape
    A = W_gen.shape[1]
    V = inp_to_act.shape[0]
    L = attn_probs.shape[1]

    b_gen2 = b_gen.reshape(1, A)
    iam2 = inp_actmask.reshape(1, A)
    om2 = out_map.reshape(1, A).astype(jnp.int32)
    wcg_pad = jnp.pad(W_cg, ((0, 0), (0, 128 - W_cg.shape[1])))
    bcg_pad = jnp.pad(b_cg, (0, 128 - b_cg.shape[0])).reshape(1, 128)

    e, s_acc, cc_acc, pg_raw = _make_stage1(B, H, A)(
        x, W_gen, b_gen2, actionmask.T, iam2, om2, wcg_pad, bcg_pad)
    gen_probs_t, out_dense_t, pog_full, wattn = _make_stage2(B, A, L)(
        e, s_acc, cc_acc, pg_raw, attn_probs)

    rows = jnp.arange(B)[:, None]
    probe_vals = jnp.broadcast_to(
        jnp.arange(V, dtype=jnp.float32)[None, :], (B, V))
    probe_idx = jnp.broadcast_to(inp_to_act[None, :], (B, V))
    winner = jnp.full((B, A), -1.0, jnp.float32).at[rows, probe_idx].set(
        probe_vals).reshape(B * A)

    od_ref = jax.new_ref(out_dense_t.reshape(A * B))
    accn = ((V + 127) // 128) * 128
    zeros_init = jnp.zeros((accn,), jnp.float32)
    _make_sc(B, A, V, L)(ctx_ids, wattn, inp_to_act, winner, zeros_init,
                         od_ref)
    out_probs = od_ref[...].reshape(A, B).T

    return (out_probs, pog_full[:, :2], gen_probs_t.T, attn_probs)


# speedup vs baseline: 1.0104x; 1.0048x over previous
import functools

import jax
import jax.numpy as jnp
from jax import lax
from jax.experimental import pallas as pl
from jax.experimental.pallas import tpu as pltpu
from jax.experimental.pallas import tpu_sc as plsc

BLK = 2048
LANE = 16
NC = 2
NS = 16
NW = NC * NS


def _make_stage1(B, H, A):
    nblk = pl.cdiv(A, BLK)

    def body(x_ref, w_ref, b_ref, am_ref, iam_ref, om_ref, wcg_ref, bcg_ref,
             e_ref, s_ref, cc_ref, pg_ref, gcol1):
        i = pl.program_id(0)
        xb = x_ref[...]
        gen = jnp.dot(xb, w_ref[...], preferred_element_type=jnp.float32)
        gen = gen + b_ref[...]

        @pl.when(i == 0)
        def _():
            pg = jnp.dot(xb, wcg_ref[...], preferred_element_type=jnp.float32)
            pg_ref[...] = pg + bcg_ref[...]
            gcol1[...] = jnp.broadcast_to(gen[:, 1:2], gcol1.shape)
            s_ref[...] = jnp.zeros_like(s_ref)
            cc_ref[...] = jnp.zeros_like(cc_ref)

        colid = i * BLK + lax.broadcasted_iota(jnp.int32, (1, BLK), 1)
        sel = jnp.where(om_ref[...] == colid, gen, gcol1[:, :1])
        valid = colid < A
        am = jnp.swapaxes(am_ref[...], 0, 1)
        e = jnp.where(valid & (am > 0), jnp.exp(sel), 0.0)
        e_ref[...] = e
        spart = jnp.sum(e, axis=1, keepdims=True)
        ccpart = jnp.sum(jnp.where(valid, am * iam_ref[...], 0), axis=1,
                         keepdims=True)
        s_ref[...] += jnp.broadcast_to(spart, s_ref.shape)
        cc_ref[...] += jnp.broadcast_to(ccpart, cc_ref.shape)

    return pl.pallas_call(
        body,
        grid=(nblk,),
        in_specs=[
            pl.BlockSpec((B, H), lambda i: (0, 0)),
            pl.BlockSpec((H, BLK), lambda i: (0, i)),
            pl.BlockSpec((1, BLK), lambda i: (0, i)),
            pl.BlockSpec((BLK, B), lambda i: (i, 0)),
            pl.BlockSpec((1, BLK), lambda i: (0, i)),
            pl.BlockSpec((1, BLK), lambda i: (0, i)),
            pl.BlockSpec((H, 128), lambda i: (0, 0)),
            pl.BlockSpec((1, 128), lambda i: (0, 0)),
        ],
        out_specs=[
            pl.BlockSpec((B, BLK), lambda i: (0, i)),
            pl.BlockSpec((B, 128), lambda i: (0, 0)),
            pl.BlockSpec((B, 128), lambda i: (0, 0)),
            pl.BlockSpec((B, 128), lambda i: (0, 0)),
        ],
        out_shape=[
            jax.ShapeDtypeStruct((B, A), jnp.float32),
            jax.ShapeDtypeStruct((B, 128), jnp.float32),
            jax.ShapeDtypeStruct((B, 128), jnp.int32),
            jax.ShapeDtypeStruct((B, 128), jnp.float32),
        ],
        scratch_shapes=[pltpu.VMEM((B, 128), jnp.float32)],
    )


def _make_stage2(B, A, L):
    nblk = pl.cdiv(A, BLK)

    def body(e_ref, s_ref, cc_ref, pg_ref, attn_ref,
             gp_ref, od_ref, pog_ref, wat_ref, p0s, invs):
        i = pl.program_id(0)

        @pl.when(i == 0)
        def _():
            s = s_ref[:, :1]
            c = cc_ref[:, :1]
            pg0 = pg_ref[:, :1]
            pg1 = pg_ref[:, 1:2]
            pg1 = jnp.where(c > 0, pg1, -jnp.inf)
            m = jnp.maximum(pg0, pg1)
            e0 = jnp.exp(pg0 - m)
            e1 = jnp.exp(pg1 - m)
            den = e0 + e1
            p0 = e0 / den
            p1 = e1 / den
            lanei = lax.broadcasted_iota(jnp.int32, (1, 128), 1)
            pog_ref[...] = jnp.where(lanei == 0, p0,
                                     jnp.where(lanei == 1, p1, 0.0))
            wat_ref[...] = p1 * attn_ref[...]
            p0s[...] = jnp.broadcast_to(p0, p0s.shape)
            invs[...] = jnp.broadcast_to(1.0 / s, invs.shape)

        gp = e_ref[...] * invs[:, :1]
        gp_ref[...] = jnp.swapaxes(gp, 0, 1)
        od_ref[...] = jnp.swapaxes(p0s[:, :1] * gp, 0, 1)

    return pl.pallas_call(
        body,
        grid=(nblk,),
        in_specs=[
            pl.BlockSpec((B, BLK), lambda i: (0, i)),
            pl.BlockSpec((B, 128), lambda i: (0, 0)),
            pl.BlockSpec((B, 128), lambda i: (0, 0)),
            pl.BlockSpec((B, 128), lambda i: (0, 0)),
            pl.BlockSpec((B, L), lambda i: (0, 0)),
        ],
        out_specs=[
            pl.BlockSpec((BLK, B), lambda i: (i, 0)),
            pl.BlockSpec((BLK, B), lambda i: (i, 0)),
            pl.BlockSpec((B, 128), lambda i: (0, 0)),
            pl.BlockSpec((B, L), lambda i: (0, 0)),
        ],
        out_shape=[
            jax.ShapeDtypeStruct((A, B), jnp.float32),
            jax.ShapeDtypeStruct((A, B), jnp.float32),
            jax.ShapeDtypeStruct((B, 128), jnp.float32),
            jax.ShapeDtypeStruct((B, L), jnp.float32),
        ],
        scratch_shapes=[pltpu.VMEM((B, 128), jnp.float32),
                        pltpu.VMEM((B, 128), jnp.float32)],
    )


def _make_sc(B, A, V, L):
    rows_per_w = B // NW
    nch = L // LANE
    accn = ((V + 127) // 128) * 128
    mesh = plsc.VectorSubcoreMesh(core_axis_name="c", subcore_axis_name="s",
                                  num_cores=NC, num_subcores=NS)

    @functools.partial(
        pl.kernel, mesh=mesh, out_type=(),
        compiler_params=pltpu.CompilerParams(needs_layout_passes=False),
        scratch_types=[
            pltpu.VMEM((L,), jnp.int32),
            pltpu.VMEM((L,), jnp.float32),
            pltpu.VMEM((L,), jnp.int32),
            pltpu.VMEM((L,), jnp.float32),
            pltpu.VMEM((L,), jnp.int32),
            pltpu.VMEM((L,), jnp.float32),
            pltpu.VMEM((L,), jnp.float32),
            pltpu.VMEM((accn,), jnp.float32),
            pltpu.SemaphoreType.DMA,
        ],
    )
    def sc_kernel(ctx_hbm, wat_hbm, i2a_hbm, win_hbm, zeros_hbm, od_hbm,
                  ctx_v, wat_v, a_v, win_v, flat_v, val_v, oval_v,
                  acc_v, sem):
        cid = lax.axis_index("c")
        sid = lax.axis_index("s")
        wid = sid * NC + cid
        zeros16 = jnp.zeros((LANE,), jnp.float32)
        pltpu.sync_copy(zeros_hbm, acc_v)
        for r in range(rows_per_w):
            b = wid * rows_per_w + r
            pltpu.sync_copy(ctx_hbm.at[b], ctx_v)
            pltpu.sync_copy(wat_hbm.at[b], wat_v)
            pltpu.async_copy(i2a_hbm.at[ctx_v], a_v, sem).wait()
            for c in range(nch):
                s = pl.ds(LANE * c, LANE)
                flat_v[s] = a_v[s] * B + b
            pltpu.async_copy(win_hbm.at[flat_v], win_v, sem).wait()
            for c in range(nch):
                s = pl.ds(LANE * c, LANE)
                plsc.addupdate_scatter(acc_v, [ctx_v[s]], wat_v[s])
            pltpu.async_copy(od_hbm.at[flat_v], oval_v, sem).wait()
            for c in range(nch):
                s = pl.ds(LANE * c, LANE)
                wv16 = win_v[s].astype(jnp.int32)
                val_v[s] = plsc.load_gather(acc_v, [wv16]) + oval_v[s]
            pltpu.sync_copy(val_v, od_hbm.at[flat_v])
            for c in range(nch):
                s = pl.ds(LANE * c, LANE)
                plsc.store_scatter(acc_v, [ctx_v[s]], zeros16)

    return sc_kernel


def kernel(x, attn_probs, ctx_ids, actionmask, W_gen, b_gen, W_cg, b_cg,
           inp_to_act, inp_actmask, out_map):
    B, H = x.shape
    A = W_gen.shape[1]
    V = inp_to_act.shape[0]
    L = attn_probs.shape[1]

    b_gen2 = b_gen.reshape(1, A)
    iam2 = inp_actmask.reshape(1, A)
    om2 = out_map.reshape(1, A).astype(jnp.int32)
    wcg_pad = jnp.pad(W_cg, ((0, 0), (0, 128 - W_cg.shape[1])))
    bcg_pad = jnp.pad(b_cg, (0, 128 - b_cg.shape[0])).reshape(1, 128)

    e, s_acc, cc_acc, pg_raw = _make_stage1(B, H, A)(
        x, W_gen, b_gen2, actionmask.T, iam2, om2, wcg_pad, bcg_pad)
    gen_probs_t, out_dense_t, pog_full, wattn = _make_stage2(B, A, L)(
        e, s_acc, cc_acc, pg_raw, attn_probs)

    rows = jnp.arange(B)[:, None]
    probe_vals = jnp.broadcast_to(
        jnp.arange(V, dtype=jnp.float32)[None, :], (B, V))
    probe_idx = jnp.broadcast_to(inp_to_act[None, :], (B, V))
    winner = jnp.full((B, A), -1.0, jnp.float32).at[rows, probe_idx].set(
        probe_vals).T.reshape(A * B)

    od_ref = jax.new_ref(out_dense_t.reshape(A * B))
    accn = ((V + 127) // 128) * 128
    zeros_init = jnp.zeros((accn,), jnp.float32)
    _make_sc(B, A, V, L)(ctx_ids, wattn, inp_to_act, winner, zeros_init,
                         od_ref)
    out_probs = od_ref[...].reshape(A, B).T

    return (out_probs, pog_full[:, :2], gen_probs_t.T, attn_probs)
